# baseline (speedup 1.0000x reference)
import functools

import jax
import jax.numpy as jnp
from jax import lax
from jax.experimental import pallas as pl
from jax.experimental.pallas import tpu as pltpu
from jax.experimental.pallas import tpu_sc as plsc

_B = 50000
_D = 128
_CHUNK = 112
_NBUF = 7
_GDEPTH = 5


@functools.cache
def _build_gather():
    info = plsc.get_sparse_core_info()
    nc, ns = info.num_cores, info.num_subcores
    nw = nc * ns
    nslot = -(-_B // (nw * _CHUNK))
    assert nslot % _NBUF == 0

    mesh = plsc.VectorSubcoreMesh(core_axis_name="c", subcore_axis_name="s")

    @functools.partial(
        pl.kernel,
        mesh=mesh,
        out_type=jax.ShapeDtypeStruct((_B, _D), jnp.float32),
        scratch_types=[
            pltpu.VMEM((nslot, _CHUNK), jnp.int32),
            pltpu.VMEM((_NBUF, _CHUNK, _D), jnp.float32),
        ] + [pltpu.SemaphoreType.DMA] * (nslot + 2 * _NBUF),
    )
    def gather_k(idx_hbm, table_hbm, out_hbm, idx_v, rows_v, *sems):
        isem = sems[0:nslot]
        gsem = sems[nslot:nslot + _NBUF]
        ssem = sems[nslot + _NBUF:nslot + 2 * _NBUF]
        wid = lax.axis_index("s") * nc + lax.axis_index("c")

        def chunk_base(s):
            return jnp.minimum((wid * nslot + s) * _CHUNK, _B - _CHUNK)

        def wait_idx(s):
            pltpu.make_async_copy(idx_hbm.at[pl.ds(0, _CHUNK)],
                                  idx_v.at[s], isem[s]).wait()

        def start_gather(s, b):
            pltpu.async_copy(table_hbm.at[idx_v.at[s]], rows_v.at[b],
                             gsem[b])

        def wait_gather(s, b):
            pltpu.make_async_copy(table_hbm.at[idx_v.at[s]], rows_v.at[b],
                                  gsem[b]).wait()

        def start_store(s, b):
            pltpu.async_copy(rows_v.at[b],
                             out_hbm.at[pl.ds(chunk_base(s), _CHUNK)],
                             ssem[b])

        def wait_store(b):
            pltpu.make_async_copy(rows_v.at[b], out_hbm.at[pl.ds(0, _CHUNK)],
                                  ssem[b]).wait()

        for s in range(nslot):
            pltpu.async_copy(idx_hbm.at[pl.ds(chunk_base(s), _CHUNK)],
                             idx_v.at[s], isem[s])
        for s in range(_GDEPTH):
            wait_idx(s)
            start_gather(s, s % _NBUF)

        for s in range(nslot):
            b = s % _NBUF
            wait_gather(s, b)
            start_store(s, b)
            s2 = s + _GDEPTH
            if s2 < nslot:
                b2 = s2 % _NBUF
                if s2 >= _NBUF:
                    wait_store(b2)
                wait_idx(s2)
                start_gather(s2, b2)

        for b in range(_NBUF):
            wait_store(b)

    return gather_k


def kernel(indices, x):
    return _build_gather()(indices.reshape(-1), x)


# speedup vs baseline: 1.0408x; 1.0408x over previous
import functools

import jax
import jax.numpy as jnp
from jax import lax
from jax.experimental import pallas as pl
from jax.experimental.pallas import tpu as pltpu
from jax.experimental.pallas import tpu_sc as plsc

_B = 50000
_D = 128
_CHUNK = 112
_NBUF = 7
_GDEPTH = 5


@functools.cache
def _build_gather():
    info = plsc.get_sparse_core_info()
    nc, ns = info.num_cores, info.num_subcores
    nw = nc * ns
    nslot = -(-_B // (nw * _CHUNK))
    assert nslot % _NBUF == 0

    mesh = plsc.VectorSubcoreMesh(core_axis_name="c", subcore_axis_name="s")

    @functools.partial(
        pl.kernel,
        mesh=mesh,
        out_type=jax.ShapeDtypeStruct((_B, _D), jnp.float32),
        scratch_types=[
            pltpu.VMEM((_NBUF, _CHUNK), jnp.int32),
            pltpu.VMEM((_NBUF, _CHUNK, _D), jnp.float32),
        ] + [pltpu.SemaphoreType.DMA] * (3 * _NBUF),
    )
    def gather_k(idx_hbm, table_hbm, out_hbm, idx_v, rows_v, *sems):
        isem = sems[0:_NBUF]
        gsem = sems[_NBUF:2 * _NBUF]
        ssem = sems[2 * _NBUF:3 * _NBUF]
        wid = lax.axis_index("s") * nc + lax.axis_index("c")

        def chunk_base(s):
            return jnp.minimum((wid + nw * s) * _CHUNK, _B - _CHUNK)

        def start_idx(s, b):
            pltpu.async_copy(idx_hbm.at[pl.ds(chunk_base(s), _CHUNK)],
                             idx_v.at[b], isem[b])

        def wait_idx(b):
            pltpu.make_async_copy(idx_hbm.at[pl.ds(0, _CHUNK)],
                                  idx_v.at[b], isem[b]).wait()

        def start_gather(b):
            pltpu.async_copy(table_hbm.at[idx_v.at[b]], rows_v.at[b],
                             gsem[b])

        def wait_gather(b):
            pltpu.make_async_copy(table_hbm.at[idx_v.at[b]], rows_v.at[b],
                                  gsem[b]).wait()

        def start_store(s, b):
            pltpu.async_copy(rows_v.at[b],
                             out_hbm.at[pl.ds(chunk_base(s), _CHUNK)],
                             ssem[b])

        def wait_store(b):
            pltpu.make_async_copy(rows_v.at[b], out_hbm.at[pl.ds(0, _CHUNK)],
                                  ssem[b]).wait()

        for b in range(_NBUF):
            start_idx(b, b)
        for b in range(_GDEPTH):
            wait_idx(b)
            start_gather(b)

        def body(j, carry):
            for b in range(_NBUF):
                s = _NBUF * j + b

                wait_gather(b)
                start_store(s, b)

                b2 = (b + _GDEPTH) % _NBUF

                @pl.when(s + _GDEPTH < nslot)
                def _():
                    @pl.when(s >= _NBUF - _GDEPTH)
                    def _():
                        wait_store(b2)
                    wait_idx(b2)
                    start_gather(b2)

                @pl.when(s + _NBUF < nslot)
                def _():
                    start_idx(s + _NBUF, b)
            return carry

        lax.fori_loop(0, nslot // _NBUF, body, None)

        for b in range(_NBUF):
            wait_store(b)

    return gather_k


def kernel(indices, x):
    return _build_gather()(indices.reshape(-1), x)
